# Initial kernel scaffold; baseline (speedup 1.0000x reference)
import jax
import jax.numpy as jnp
from jax.experimental import pallas as pl


def kernel(prefix, embedding, w1, b1, w2, b2):
    raise NotImplementedError("write your pallas kernel here")


# speedup vs baseline: 1.3064x; 1.3064x over previous
import functools

import jax
import jax.numpy as jnp
from jax.experimental import pallas as pl
from jax.experimental.pallas import tpu as pltpu


def _ceil_to(x: int, m: int) -> int:
    return ((x + m - 1) // m) * m


def _hidden_kernel(emb_ref, w1_ref, b1_ref, h_ref):
    h_ref[...] = jnp.tanh(
        jnp.dot(emb_ref[...], w1_ref[...], preferred_element_type=jnp.float32)
        + b1_ref[...]
    ).astype(h_ref.dtype)


def _table_kernel(h_ref, w2_ref, b2_ref, t_ref):
    acc = jnp.dot(
        h_ref[...].astype(jnp.float32), w2_ref[...],
        preferred_element_type=jnp.float32,
    )
    t_ref[...] = (acc + b2_ref[...]).astype(t_ref.dtype)


def _onehot_gather_kernel(idx_ref, t_ref, out_ref):
    idx = idx_ref[...]
    tile_n = idx.shape[0]
    p_pad = t_ref.shape[0]
    col = jax.lax.broadcasted_iota(jnp.int32, (tile_n, p_pad), 1)
    onehot = (col == idx).astype(jnp.bfloat16)
    out_ref[...] = jnp.dot(
        onehot, t_ref[...], preferred_element_type=jnp.float32
    )


@functools.partial(jax.jit, static_argnames=("tile_n",))
def _prefix_encoder(prefix, embedding, w1, b1, w2, b2, *, tile_n=1024):
    B, L = prefix.shape
    P, H = embedding.shape
    N = B * L

    f32 = jnp.float32
    bf16 = jnp.bfloat16

    h_pad = _ceil_to(H, 256)
    p_pad = _ceil_to(P, 8)
    hc = h_pad // 2
    tile_n = min(tile_n, _ceil_to(N, 8))
    n_tiles = pl.cdiv(N, tile_n)
    n_pad = n_tiles * tile_n

    emb_p = jnp.pad(embedding.astype(f32), ((0, p_pad - P), (0, h_pad - H)))
    w1_p = jnp.pad(w1.astype(f32), ((0, h_pad - H), (0, h_pad - H)))
    w2_p = jnp.pad(w2.astype(f32), ((0, h_pad - H), (0, h_pad - H)))
    b1_p = jnp.pad(b1.astype(f32), (0, h_pad - H)).reshape(1, h_pad)
    b2_p = jnp.pad(b2.astype(f32), (0, h_pad - H)).reshape(1, h_pad)

    hidden = pl.pallas_call(
        _hidden_kernel,
        out_shape=jax.ShapeDtypeStruct((p_pad, h_pad), bf16),
        grid=(2,),
        in_specs=[
            pl.BlockSpec((p_pad, h_pad), lambda j: (0, 0)),
            pl.BlockSpec((h_pad, hc), lambda j: (0, j)),
            pl.BlockSpec((1, hc), lambda j: (0, j)),
        ],
        out_specs=pl.BlockSpec((p_pad, hc), lambda j: (0, j)),
        compiler_params=pltpu.CompilerParams(
            dimension_semantics=("parallel",),
            vmem_limit_bytes=48 * 1024 * 1024),
        cost_estimate=pl.CostEstimate(
            flops=2 * p_pad * h_pad * h_pad,
            transcendentals=p_pad * h_pad,
            bytes_accessed=4 * (p_pad * h_pad + h_pad * h_pad + h_pad)
            + 2 * p_pad * h_pad),
    )(emb_p, w1_p, b1_p)

    table = pl.pallas_call(
        _table_kernel,
        out_shape=jax.ShapeDtypeStruct((p_pad, h_pad), bf16),
        grid=(2,),
        in_specs=[
            pl.BlockSpec((p_pad, h_pad), lambda j: (0, 0)),
            pl.BlockSpec((h_pad, hc), lambda j: (0, j)),
            pl.BlockSpec((1, hc), lambda j: (0, j)),
        ],
        out_specs=pl.BlockSpec((p_pad, hc), lambda j: (0, j)),
        compiler_params=pltpu.CompilerParams(
            dimension_semantics=("parallel",),
            vmem_limit_bytes=48 * 1024 * 1024),
        cost_estimate=pl.CostEstimate(
            flops=2 * p_pad * h_pad * h_pad,
            transcendentals=0,
            bytes_accessed=4 * (h_pad * h_pad + h_pad)
            + 2 * (2 * p_pad * h_pad)),
    )(hidden, w2_p, b2_p)

    idx = jnp.pad(prefix.reshape(N).astype(jnp.int32), (0, n_pad - N))
    idx = idx.reshape(n_pad, 1)

    out = pl.pallas_call(
        _onehot_gather_kernel,
        out_shape=jax.ShapeDtypeStruct((n_pad, h_pad), f32),
        grid=(n_tiles,),
        in_specs=[
            pl.BlockSpec((tile_n, 1), lambda i: (i, 0)),
            pl.BlockSpec((p_pad, h_pad), lambda i: (0, 0)),
        ],
        out_specs=pl.BlockSpec((tile_n, h_pad), lambda i: (i, 0)),
        compiler_params=pltpu.CompilerParams(
            dimension_semantics=("parallel",),
            vmem_limit_bytes=48 * 1024 * 1024),
        cost_estimate=pl.CostEstimate(
            flops=2 * n_pad * p_pad * h_pad,
            transcendentals=0,
            bytes_accessed=4 * n_pad * (1 + h_pad) + 2 * p_pad * h_pad),
    )(idx, table)

    return out[:N, :H].reshape(B, L, H)


def kernel(prefix, embedding, w1, b1, w2, b2):
    return _prefix_encoder(prefix, embedding, w1, b1, w2, b2)


# speedup vs baseline: 1.3065x; 1.0001x over previous
import functools

import jax
import jax.numpy as jnp
from jax.experimental import pallas as pl
from jax.experimental.pallas import tpu as pltpu


def _ceil_to(x: int, m: int) -> int:
    return ((x + m - 1) // m) * m


def _hidden_kernel(emb_ref, w1_ref, b1_ref, h_ref):
    h_ref[...] = jnp.tanh(
        jnp.dot(emb_ref[...], w1_ref[...], preferred_element_type=jnp.float32)
        + b1_ref[...]
    ).astype(h_ref.dtype)


def _table_kernel(h_ref, w2_ref, b2_ref, t_ref):
    acc = jnp.dot(
        h_ref[...].astype(jnp.float32), w2_ref[...],
        preferred_element_type=jnp.float32,
    )
    t_ref[...] = (acc + b2_ref[...]).astype(t_ref.dtype)


def _onehot_gather_kernel(idx_ref, t_ref, out_ref):
    idx = idx_ref[...]
    tile_n = idx.shape[0]
    p_pad = t_ref.shape[0]
    col = jax.lax.broadcasted_iota(jnp.int32, (tile_n, p_pad), 1)
    onehot = (col == idx).astype(jnp.bfloat16)
    out_ref[...] = jnp.dot(
        onehot, t_ref[...], preferred_element_type=jnp.float32
    )


@functools.partial(jax.jit, static_argnames=("tile_n",))
def _prefix_encoder(prefix, embedding, w1, b1, w2, b2, *, tile_n=2048):
    B, L = prefix.shape
    P, H = embedding.shape
    N = B * L

    f32 = jnp.float32
    bf16 = jnp.bfloat16

    h_pad = _ceil_to(H, 256)
    p_pad = _ceil_to(P, 8)
    hc = h_pad // 2
    tile_n = min(tile_n, _ceil_to(N, 8))
    n_tiles = pl.cdiv(N, tile_n)
    n_pad = n_tiles * tile_n

    emb_p = jnp.pad(embedding.astype(f32), ((0, p_pad - P), (0, h_pad - H)))
    w1_p = jnp.pad(w1.astype(f32), ((0, h_pad - H), (0, h_pad - H)))
    w2_p = jnp.pad(w2.astype(f32), ((0, h_pad - H), (0, h_pad - H)))
    b1_p = jnp.pad(b1.astype(f32), (0, h_pad - H)).reshape(1, h_pad)
    b2_p = jnp.pad(b2.astype(f32), (0, h_pad - H)).reshape(1, h_pad)

    hidden = pl.pallas_call(
        _hidden_kernel,
        out_shape=jax.ShapeDtypeStruct((p_pad, h_pad), bf16),
        grid=(2,),
        in_specs=[
            pl.BlockSpec((p_pad, h_pad), lambda j: (0, 0)),
            pl.BlockSpec((h_pad, hc), lambda j: (0, j)),
            pl.BlockSpec((1, hc), lambda j: (0, j)),
        ],
        out_specs=pl.BlockSpec((p_pad, hc), lambda j: (0, j)),
        compiler_params=pltpu.CompilerParams(
            dimension_semantics=("parallel",),
            vmem_limit_bytes=48 * 1024 * 1024),
        cost_estimate=pl.CostEstimate(
            flops=2 * p_pad * h_pad * h_pad,
            transcendentals=p_pad * h_pad,
            bytes_accessed=4 * (p_pad * h_pad + h_pad * h_pad + h_pad)
            + 2 * p_pad * h_pad),
    )(emb_p, w1_p, b1_p)

    table = pl.pallas_call(
        _table_kernel,
        out_shape=jax.ShapeDtypeStruct((p_pad, h_pad), bf16),
        grid=(2,),
        in_specs=[
            pl.BlockSpec((p_pad, h_pad), lambda j: (0, 0)),
            pl.BlockSpec((h_pad, hc), lambda j: (0, j)),
            pl.BlockSpec((1, hc), lambda j: (0, j)),
        ],
        out_specs=pl.BlockSpec((p_pad, hc), lambda j: (0, j)),
        compiler_params=pltpu.CompilerParams(
            dimension_semantics=("parallel",),
            vmem_limit_bytes=48 * 1024 * 1024),
        cost_estimate=pl.CostEstimate(
            flops=2 * p_pad * h_pad * h_pad,
            transcendentals=0,
            bytes_accessed=4 * (h_pad * h_pad + h_pad)
            + 2 * (2 * p_pad * h_pad)),
    )(hidden, w2_p, b2_p)

    idx = jnp.pad(prefix.reshape(N).astype(jnp.int32), (0, n_pad - N))
    idx = idx.reshape(n_pad, 1)

    out = pl.pallas_call(
        _onehot_gather_kernel,
        out_shape=jax.ShapeDtypeStruct((n_pad, h_pad), f32),
        grid=(n_tiles,),
        in_specs=[
            pl.BlockSpec((tile_n, 1), lambda i: (i, 0)),
            pl.BlockSpec((p_pad, h_pad), lambda i: (0, 0)),
        ],
        out_specs=pl.BlockSpec((tile_n, h_pad), lambda i: (i, 0)),
        compiler_params=pltpu.CompilerParams(
            dimension_semantics=("parallel",),
            vmem_limit_bytes=48 * 1024 * 1024),
        cost_estimate=pl.CostEstimate(
            flops=2 * n_pad * p_pad * h_pad,
            transcendentals=0,
            bytes_accessed=4 * n_pad * (1 + h_pad) + 2 * p_pad * h_pad),
    )(idx, table)

    return out[:N, :H].reshape(B, L, H)


def kernel(prefix, embedding, w1, b1, w2, b2):
    return _prefix_encoder(prefix, embedding, w1, b1, w2, b2)
